# baseline (speedup 1.0000x reference)
import jax
import jax.numpy as jnp
from jax.experimental import pallas as pl
from jax.experimental.pallas import tpu as pltpu

_BM = 512


def _mm_kernel(x_ref, w_ref, o_ref, wb_ref):
    i = pl.program_id(0)

    @pl.when(i == 0)
    def _():
        wb_ref[...] = w_ref[...].astype(jnp.bfloat16)

    @pl.when(i > 0)
    def _():
        o_ref[...] = jax.lax.dot_general(
            x_ref[...].astype(jnp.bfloat16), wb_ref[...],
            dimension_numbers=(((1,), (0,)), ((), ())),
            preferred_element_type=jnp.float32,
        )


def kernel(input, W):
    B, M, K = input.shape
    N = W.shape[1]
    x2 = input.reshape(B * M, K)

    def _xo_index(i):
        return (jnp.where(i == 0, 0, i - 1), 0)

    out = pl.pallas_call(
        _mm_kernel,
        grid=(B * M // _BM + 1,),
        in_specs=[
            pl.BlockSpec((_BM, K), _xo_index),
            pl.BlockSpec((K, N), lambda i: (0, 0)),
        ],
        out_specs=pl.BlockSpec((_BM, N), _xo_index),
        out_shape=jax.ShapeDtypeStruct((B * M, N), jnp.float32),
        scratch_shapes=[pltpu.VMEM((K, N), jnp.bfloat16)],
        compiler_params=pltpu.CompilerParams(
            dimension_semantics=("arbitrary",),
        ),
    )(x2, W)
    return out.reshape(B, M, N)
